# Initial kernel scaffold; baseline (speedup 1.0000x reference)
import functools

import jax
import jax.numpy as jnp
from jax import lax
from jax.experimental import pallas as pl
from jax.experimental.pallas import tpu as pltpu
from jax.experimental.pallas import tpu_sc as plsc

N = 10000
E = 320000
D = 128
DH = D // 2
N_PAD = 10240
E_F = E + N
E_PAD = 331776
C = 128
NCH = E_PAD // 16 // C
EP32 = E_PAD // 32
NPT = N_PAD // 16

_MESH = plsc.VectorSubcoreMesh(core_axis_name="c", subcore_axis_name="s")


@functools.partial(
    pl.kernel,
    out_type=jax.ShapeDtypeStruct((32, N_PAD), jnp.float32),
    mesh=_MESH,
    scratch_types=[
        pltpu.VMEM((EP32,), jnp.int32),
        pltpu.VMEM((EP32,), jnp.float32),
        pltpu.VMEM((N_PAD,), jnp.float32),
    ],
)
def _deg_kernel(dst_hbm, ew_hbm, out_hbm, dstv, ewv, deg):
    wid = lax.axis_index("s") * 2 + lax.axis_index("c")
    pltpu.sync_copy(dst_hbm.at[wid], dstv)
    pltpu.sync_copy(ew_hbm.at[wid], ewv)

    def zero_body(i, _):
        deg[pl.ds(i * 16, 16)] = jnp.zeros((16,), jnp.float32)
        return 0

    lax.fori_loop(0, N_PAD // 16, zero_body, 0)

    def acc_body(e, _):
        d = dstv[e]
        deg[d] = deg[d] + ewv[e]
        return 0

    lax.fori_loop(0, EP32, acc_body, 0)
    pltpu.sync_copy(deg, out_hbm.at[wid])


def _dinv_body(degp_ref, out_ref):
    d = jnp.sum(degp_ref[...], axis=0)
    out_ref[...] = jnp.where(d > 0.0, lax.rsqrt(d), 0.0)


def _dinv(deg_partial):
    return pl.pallas_call(
        _dinv_body,
        out_shape=jax.ShapeDtypeStruct((N_PAD // 128, 128), jnp.float32),
    )(deg_partial)


@functools.partial(
    pl.kernel,
    out_type=jax.ShapeDtypeStruct((32, EP32), jnp.float32),
    mesh=_MESH,
    scratch_types=[
        pltpu.VMEM((N_PAD,), jnp.float32),
        pltpu.VMEM((EP32,), jnp.int32),
        pltpu.VMEM((EP32,), jnp.int32),
        pltpu.VMEM((EP32,), jnp.float32),
        pltpu.VMEM((EP32,), jnp.float32),
    ],
)
def _norm_kernel(dinv_hbm, src_hbm, dst_hbm, ew_hbm, out_hbm,
                 dinvv, srcv, dstv, ewv, normv):
    wid = lax.axis_index("s") * 2 + lax.axis_index("c")
    pltpu.sync_copy(dinv_hbm, dinvv)
    pltpu.sync_copy(src_hbm.at[wid], srcv)
    pltpu.sync_copy(dst_hbm.at[wid], dstv)
    pltpu.sync_copy(ew_hbm.at[wid], ewv)

    def body(i, _):
        sl = pl.ds(i * 16, 16)
        sv = plsc.load_gather(dinvv, [srcv[sl]])
        tv = plsc.load_gather(dinvv, [dstv[sl]])
        normv[sl] = sv * tv * ewv[sl]
        return 0

    lax.fori_loop(0, EP32 // 16, body, 0)
    pltpu.sync_copy(normv, out_hbm.at[wid])


@functools.partial(
    pl.kernel,
    out_type=[
        jax.ShapeDtypeStruct((N_PAD, DH), jnp.float32),
        jax.ShapeDtypeStruct((N_PAD, DH), jnp.float32),
    ],
    mesh=_MESH,
    scratch_types=[
        pltpu.VMEM((NCH, C), jnp.int32),
        pltpu.VMEM((NCH, C), jnp.int32),
        pltpu.VMEM((NCH, C), jnp.float32),
        pltpu.VMEM((C, DH), jnp.float32),
        pltpu.VMEM((C, DH), jnp.float32),
        pltpu.VMEM_SHARED((N_PAD, DH), jnp.float32),
        pltpu.SemaphoreType.DMA,
    ],
)
def _hop_kernel(h0_hbm, h1_hbm, src_hbm, dst_hbm, norm_hbm, out0_hbm, out1_hbm,
                srcv, dstv, normv, rows, zbuf, acc, gsem):
    s = lax.axis_index("s")
    c = lax.axis_index("c")
    pltpu.sync_copy(src_hbm.at[s], srcv)
    pltpu.sync_copy(dst_hbm.at[s], dstv)
    pltpu.sync_copy(norm_hbm.at[s], normv)

    def zb(i, _):
        for q in range(DH // 16):
            zbuf[i, pl.ds(q * 16, 16)] = jnp.zeros((16,), jnp.float32)
        return 0

    lax.fori_loop(0, C, zb, 0)
    for t in range(NPT // C):
        pltpu.sync_copy(zbuf, acc.at[pl.ds(s * NPT + t * C, C)])
    plsc.subcore_barrier()

    def run(h_hbm):
        def chunk(j, _):
            pltpu.async_copy(h_hbm.at[srcv.at[j]], rows, gsem).wait()

            def scale(e, _):
                sc = normv[j, e]
                bc = lax.broadcast(sc, (16,))
                for q in range(DH // 16):
                    sl = pl.ds(q * 16, 16)
                    rows[e, sl] = rows[e, sl] * bc
                return 0

            lax.fori_loop(0, C, scale, 0)
            pltpu.sync_copy(rows, acc.at[dstv.at[j]], add=True)
            return 0

        lax.fori_loop(0, NCH, chunk, 0)

    @pl.when(c == 0)
    def _():
        run(h0_hbm)

    @pl.when(c == 1)
    def _():
        run(h1_hbm)

    plsc.subcore_barrier()
    nsl = pl.ds(s * NPT, NPT)

    @pl.when(c == 0)
    def _():
        pltpu.sync_copy(acc.at[nsl], out0_hbm.at[nsl])

    @pl.when(c == 1)
    def _():
        pltpu.sync_copy(acc.at[nsl], out1_hbm.at[nsl])


def _head_body(h0_ref, h1_ref, w_ref, b_ref, out_ref):
    w = w_ref[...]
    z = lax.dot_general(h0_ref[...], w[:, :DH], (((1,), (1,)), ((), ())),
                        preferred_element_type=jnp.float32)
    z = z + lax.dot_general(h1_ref[...], w[:, DH:], (((1,), (1,)), ((), ())),
                            preferred_element_type=jnp.float32)
    z = z + b_ref[...]
    z = jnp.maximum(z, 0.0)
    m = jnp.max(z, axis=-1, keepdims=True)
    zs = z - m
    lse = jnp.log(jnp.sum(jnp.exp(zs), axis=-1, keepdims=True))
    out_ref[...] = zs - lse


def _head(h0, h1, W, b2):
    blk = 1024
    grid = (N_PAD // blk,)
    return pl.pallas_call(
        _head_body,
        grid=grid,
        in_specs=[
            pl.BlockSpec((blk, DH), lambda i: (i, 0)),
            pl.BlockSpec((blk, DH), lambda i: (i, 0)),
            pl.BlockSpec((D, D), lambda i: (0, 0)),
            pl.BlockSpec((1, D), lambda i: (0, 0)),
        ],
        out_specs=pl.BlockSpec((blk, D), lambda i: (i, 0)),
        out_shape=jax.ShapeDtypeStruct((N_PAD, D), jnp.float32),
    )(h0, h1, W, b2)


def kernel(x, edge_index, edge_attr, W, b):
    src = edge_index[0].astype(jnp.int32)
    dst = edge_index[1].astype(jnp.int32)
    loop = jnp.arange(N, dtype=jnp.int32)
    pad = E_PAD - E_F
    srcf = jnp.concatenate([src, loop, jnp.zeros((pad,), jnp.int32)])
    dstf = jnp.concatenate([dst, loop, jnp.zeros((pad,), jnp.int32)])
    ewf = jnp.concatenate([edge_attr.astype(jnp.float32),
                           jnp.ones((N,), jnp.float32),
                           jnp.zeros((pad,), jnp.float32)])

    src32 = srcf.reshape(32, EP32)
    dst32 = dstf.reshape(32, EP32)
    ew32 = ewf.reshape(32, EP32)

    degp = _deg_kernel(dst32, ew32)
    dinv = _dinv(degp.reshape(32, N_PAD // 128, 128)).reshape(N_PAD)
    norm = _norm_kernel(dinv, src32, dst32, ew32)

    src16 = srcf.reshape(16, NCH, C)
    dst16 = dstf.reshape(16, NCH, C)
    norm16 = norm.reshape(16, NCH, C)

    xp = jnp.pad(x.astype(jnp.float32), ((0, N_PAD - N), (0, 0)))
    h0 = xp[:, :DH]
    h1 = xp[:, DH:]
    for _ in range(2):
        h0, h1 = _hop_kernel(h0, h1, src16, dst16, norm16)

    out = _head(h0, h1, W.astype(jnp.float32),
                b.astype(jnp.float32).reshape(1, D))
    return out[:N]


# speedup vs baseline: 5.7192x; 5.7192x over previous
import functools

import jax
import jax.numpy as jnp
from jax import lax
from jax.experimental import pallas as pl
from jax.experimental.pallas import tpu as pltpu
from jax.experimental.pallas import tpu_sc as plsc

N = 10000
E = 320000
D = 128
N_PAD = 10240
E_PAD = 327680
C = 128
EP32 = E_PAD // 32
NCH32 = EP32 // C
NPT = N_PAD // 16

_MESH = plsc.VectorSubcoreMesh(core_axis_name="c", subcore_axis_name="s")
_SC_PARAMS = pltpu.CompilerParams(needs_layout_passes=False)


@functools.partial(
    pl.kernel,
    out_type=jax.ShapeDtypeStruct((32, N_PAD), jnp.float32),
    mesh=_MESH,
    compiler_params=_SC_PARAMS,
    scratch_types=[
        pltpu.VMEM((EP32,), jnp.int32),
        pltpu.VMEM((EP32,), jnp.float32),
        pltpu.VMEM((N_PAD,), jnp.float32),
    ],
)
def _deg_kernel(dst_hbm, ew_hbm, out_hbm, dstv, ewv, deg):
    wid = lax.axis_index("s") * 2 + lax.axis_index("c")
    pltpu.sync_copy(dst_hbm.at[wid], dstv)
    pltpu.sync_copy(ew_hbm.at[wid], ewv)

    def zero_body(i, _):
        deg[pl.ds(i * 16, 16)] = jnp.zeros((16,), jnp.float32)
        return 0

    lax.fori_loop(0, N_PAD // 16, zero_body, 0)

    lane = lax.iota(jnp.int32, 16)

    def acc_body(i, _):
        sl = pl.ds(i * 16, 16)
        dv = dstv[sl]
        wv = ewv[sl]
        for l in range(16):
            plsc.addupdate_scatter(deg, [dv], wv, mask=lane == l)
        return 0

    lax.fori_loop(0, EP32 // 16, acc_body, 0)
    pltpu.sync_copy(deg, out_hbm.at[wid])


def _dinv_body(degp_ref, out_ref):
    d = jnp.sum(degp_ref[...], axis=0) + 1.0
    out_ref[...] = lax.rsqrt(d)


def _dinv(degp):
    return pl.pallas_call(
        _dinv_body,
        out_shape=jax.ShapeDtypeStruct((N_PAD // 128, 128), jnp.float32),
    )(degp)


@functools.partial(
    pl.kernel,
    out_type=jax.ShapeDtypeStruct((32, EP32), jnp.float32),
    mesh=_MESH,
    compiler_params=_SC_PARAMS,
    scratch_types=[
        pltpu.VMEM((N_PAD,), jnp.float32),
        pltpu.VMEM((EP32,), jnp.int32),
        pltpu.VMEM((EP32,), jnp.int32),
        pltpu.VMEM((EP32,), jnp.float32),
        pltpu.VMEM((EP32,), jnp.float32),
    ],
)
def _norm_kernel(dinv_hbm, src_hbm, dst_hbm, ew_hbm, out_hbm,
                 dinvv, srcv, dstv, ewv, normv):
    wid = lax.axis_index("s") * 2 + lax.axis_index("c")
    pltpu.sync_copy(dinv_hbm, dinvv)
    pltpu.sync_copy(src_hbm.at[wid], srcv)
    pltpu.sync_copy(dst_hbm.at[wid], dstv)
    pltpu.sync_copy(ew_hbm.at[wid], ewv)

    def body(i, _):
        sl = pl.ds(i * 16, 16)
        sv = plsc.load_gather(dinvv, [srcv[sl]])
        tv = plsc.load_gather(dinvv, [dstv[sl]])
        normv[sl] = sv * tv * ewv[sl]
        return 0

    lax.fori_loop(0, EP32 // 16, body, 0)
    pltpu.sync_copy(normv, out_hbm.at[wid])


NCH16 = E_PAD // 16 // C
NCHQ = NCH16 // 4
NHALF = N_PAD // 2
NACC = NHALF + C
NOUT = NHALF // 16


@functools.partial(
    pl.kernel,
    out_type=jax.ShapeDtypeStruct((N_PAD, D), jnp.float32),
    mesh=_MESH,
    compiler_params=_SC_PARAMS,
    scratch_types=[
        pltpu.VMEM((NCHQ, C), jnp.int32),
        pltpu.VMEM((NCHQ, C), jnp.int32),
        pltpu.VMEM((NCHQ, C), jnp.float32),
        pltpu.VMEM((C, D), jnp.float32),
        pltpu.VMEM((C, D), jnp.float32),
        pltpu.VMEM_SHARED((NACC, D), jnp.float32),
        pltpu.SemaphoreType.DMA,
        pltpu.SemaphoreType.DMA,
        pltpu.SemaphoreType.DMA,
        pltpu.SemaphoreType.DMA,
    ],
)
def _hop_kernel(h_hbm, src_hbm, dst_hbm, norm_hbm, out_hbm,
                srcv, dstv, normv, rows0, rows1, acc, gs0, gs1, ss0, ss1):
    s = lax.axis_index("s")
    c = lax.axis_index("c")
    base = c * NHALF

    def zb(i, _):
        for q in range(D // 16):
            rows0[i, pl.ds(q * 16, 16)] = jnp.zeros((16,), jnp.float32)
        return 0

    lax.fori_loop(0, C, zb, 0)
    for t in range(NOUT // C):
        pltpu.sync_copy(rows0, acc.at[pl.ds(s * NOUT + t * C, C)])
    plsc.subcore_barrier()

    def scale(rows, j):
        def sg(g, _):
            nv = normv[j, pl.ds(g * 16, 16)]
            for l in range(16):
                bc = lax.broadcast(nv[l], (16,))
                e = g * 16 + l
                for q in range(D // 16):
                    sl = pl.ds(q * 16, 16)
                    rows[e, sl] = rows[e, sl] * bc
            return 0

        lax.fori_loop(0, C // 16, sg, 0)

    NPAIR = NCHQ // 2
    for qtr in range(4):
        pltpu.sync_copy(src_hbm.at[s * 4 + qtr], srcv)
        pltpu.sync_copy(dst_hbm.at[s * 4 + qtr], dstv)
        pltpu.sync_copy(norm_hbm.at[s * 4 + qtr], normv)

        def remap(j, _):
            def rg(g, _):
                sl = pl.ds(g * 16, 16)
                dv = dstv[j, sl] - base
                ok = (dv >= 0) & (dv < NHALF)
                dstv[j, sl] = jnp.where(ok, dv, NHALF)
                return 0

            lax.fori_loop(0, C // 16, rg, 0)
            return 0

        lax.fori_loop(0, NCHQ, remap, 0)

        pltpu.async_copy(h_hbm.at[srcv.at[0]], rows0, gs0)

        def pair(j2, _):
            a = j2 * 2
            pltpu.make_async_copy(h_hbm.at[srcv.at[a]], rows0, gs0).wait()

            @pl.when(j2 > 0)
            def _():
                pltpu.make_async_copy(rows1, acc.at[dstv.at[a - 1]], ss1).wait()

            pltpu.async_copy(h_hbm.at[srcv.at[a + 1]], rows1, gs1)
            scale(rows0, a)
            pltpu.async_copy(rows0, acc.at[dstv.at[a]], ss0, add=True)
            pltpu.make_async_copy(h_hbm.at[srcv.at[a + 1]], rows1, gs1).wait()

            @pl.when(j2 < NPAIR - 1)
            def _():
                pltpu.make_async_copy(rows0, acc.at[dstv.at[a]], ss0).wait()
                pltpu.async_copy(h_hbm.at[srcv.at[a + 2]], rows0, gs0)

            scale(rows1, a + 1)
            pltpu.async_copy(rows1, acc.at[dstv.at[a + 1]], ss1, add=True)
            return 0

        lax.fori_loop(0, NPAIR, pair, 0)
        pltpu.make_async_copy(rows0, acc.at[dstv.at[NCHQ - 2]], ss0).wait()
        pltpu.make_async_copy(rows1, acc.at[dstv.at[NCHQ - 1]], ss1).wait()

    plsc.subcore_barrier()
    pltpu.sync_copy(acc.at[pl.ds(s * NOUT, NOUT)],
                    out_hbm.at[pl.ds(base + s * NOUT, NOUT)])


ER = E // 128
EPR = E_PAD // 128


def _pad_body(s_ref, d_ref, w_ref, so_ref, do_ref, wo_ref):
    so_ref[...] = jnp.zeros_like(so_ref)
    do_ref[...] = jnp.zeros_like(do_ref)
    wo_ref[...] = jnp.zeros_like(wo_ref)
    so_ref[pl.ds(0, ER), :] = s_ref[...]
    do_ref[pl.ds(0, ER), :] = d_ref[...]
    wo_ref[pl.ds(0, ER), :] = w_ref[...]


def _pad_edges(s, d, w):
    return pl.pallas_call(
        _pad_body,
        out_shape=[
            jax.ShapeDtypeStruct((EPR, 128), jnp.int32),
            jax.ShapeDtypeStruct((EPR, 128), jnp.int32),
            jax.ShapeDtypeStruct((EPR, 128), jnp.float32),
        ],
    )(s.reshape(ER, 128), d.reshape(ER, 128), w.reshape(ER, 128))


def _combine_body(a_ref, dv_ref, h_ref, out_ref):
    dv = dv_ref[...]
    out_ref[...] = a_ref[...] + (dv * dv) * h_ref[...]


def _combine(a, dinv_col, h):
    blk = 2000
    return pl.pallas_call(
        _combine_body,
        grid=(N // blk,),
        in_specs=[
            pl.BlockSpec((blk, D), lambda i: (i, 0)),
            pl.BlockSpec((blk, 1), lambda i: (i, 0)),
            pl.BlockSpec((blk, D), lambda i: (i, 0)),
        ],
        out_specs=pl.BlockSpec((blk, D), lambda i: (i, 0)),
        out_shape=jax.ShapeDtypeStruct((N, D), jnp.float32),
    )(a, dinv_col, h)


def _head_body(q_ref, dv_ref, h1_ref, w_ref, b_ref, out_ref):
    dv = dv_ref[...]
    h = q_ref[...] + (dv * dv) * h1_ref[...]
    z = lax.dot_general(h, w_ref[...], (((1,), (1,)), ((), ())),
                        preferred_element_type=jnp.float32)
    z = z + b_ref[...]
    z = jnp.maximum(z, 0.0)
    m = jnp.max(z, axis=-1, keepdims=True)
    zs = z - m
    lse = jnp.log(jnp.sum(jnp.exp(zs), axis=-1, keepdims=True))
    out_ref[...] = zs - lse


def _head(q, dinv_col, h1, W, b2):
    blk = 2000
    return pl.pallas_call(
        _head_body,
        grid=(N // blk,),
        in_specs=[
            pl.BlockSpec((blk, D), lambda i: (i, 0)),
            pl.BlockSpec((blk, 1), lambda i: (i, 0)),
            pl.BlockSpec((blk, D), lambda i: (i, 0)),
            pl.BlockSpec((D, D), lambda i: (0, 0)),
            pl.BlockSpec((1, D), lambda i: (0, 0)),
        ],
        out_specs=pl.BlockSpec((blk, D), lambda i: (i, 0)),
        out_shape=jax.ShapeDtypeStruct((N, D), jnp.float32),
    )(q, dinv_col, h1, W, b2)


def kernel(x, edge_index, edge_attr, W, b):
    srcf, dstf, ewf = _pad_edges(edge_index[0].astype(jnp.int32),
                                 edge_index[1].astype(jnp.int32),
                                 edge_attr.astype(jnp.float32))

    src32 = srcf.reshape(32, EP32)
    dst32 = dstf.reshape(32, EP32)
    ew32 = ewf.reshape(32, EP32)

    degp = _deg_kernel(dst32, ew32)
    dinv = _dinv(degp.reshape(32, N_PAD // 128, 128)).reshape(N_PAD)
    norm = _norm_kernel(dinv, src32, dst32, ew32)

    srcc = srcf.reshape(64, NCHQ, C)
    dstc = dstf.reshape(64, NCHQ, C)
    normc = norm.reshape(64, NCHQ, C)

    dinv_col = dinv.reshape(N_PAD, 1)[:N]
    h = x.astype(jnp.float32)
    p = _hop_kernel(h, srcc, dstc, normc)
    h1 = _combine(p[:N], dinv_col, h)
    q = _hop_kernel(h1, srcc, dstc, normc)

    return _head(q[:N], dinv_col, h1, W.astype(jnp.float32),
                 b.astype(jnp.float32).reshape(1, D))


# speedup vs baseline: 8.0986x; 1.4160x over previous
import functools

import jax
import jax.numpy as jnp
from jax import lax
from jax.experimental import pallas as pl
from jax.experimental.pallas import tpu as pltpu
from jax.experimental.pallas import tpu_sc as plsc

N = 10000
E = 320000
D = 128
N_PAD = 10240
E_PAD = 323584
C = 64
EP32 = E_PAD // 32
NCH32 = EP32 // C
NPT = N_PAD // 16

_MESH = plsc.VectorSubcoreMesh(core_axis_name="c", subcore_axis_name="s")
_SC_PARAMS = pltpu.CompilerParams(needs_layout_passes=False)


@functools.partial(
    pl.kernel,
    out_type=jax.ShapeDtypeStruct((32, N_PAD), jnp.float32),
    mesh=_MESH,
    compiler_params=_SC_PARAMS,
    scratch_types=[
        pltpu.VMEM((EP32,), jnp.int32),
        pltpu.VMEM((EP32,), jnp.float32),
        pltpu.VMEM((N_PAD,), jnp.float32),
    ],
)
def _deg_kernel(dst_hbm, ew_hbm, out_hbm, dstv, ewv, deg):
    wid = lax.axis_index("s") * 2 + lax.axis_index("c")
    pltpu.sync_copy(dst_hbm.at[wid], dstv)
    pltpu.sync_copy(ew_hbm.at[wid], ewv)

    def zero_body(i, _):
        deg[pl.ds(i * 16, 16)] = jnp.zeros((16,), jnp.float32)
        return 0

    lax.fori_loop(0, N_PAD // 16, zero_body, 0)

    lane = lax.iota(jnp.int32, 16)

    def acc_body(i, _):
        sl = pl.ds(i * 16, 16)
        dv = dstv[sl]
        wv = ewv[sl]
        for l in range(16):
            plsc.addupdate_scatter(deg, [dv], wv, mask=lane == l)
        return 0

    lax.fori_loop(0, EP32 // 16, acc_body, 0)
    pltpu.sync_copy(deg, out_hbm.at[wid])


def _dinv_body(degp_ref, out_ref):
    d = jnp.sum(degp_ref[...], axis=0) + 1.0
    out_ref[...] = lax.rsqrt(d)


def _dinv(degp):
    return pl.pallas_call(
        _dinv_body,
        out_shape=jax.ShapeDtypeStruct((N_PAD // 128, 128), jnp.float32),
    )(degp)


@functools.partial(
    pl.kernel,
    out_type=jax.ShapeDtypeStruct((32, EP32), jnp.float32),
    mesh=_MESH,
    compiler_params=_SC_PARAMS,
    scratch_types=[
        pltpu.VMEM((N_PAD,), jnp.float32),
        pltpu.VMEM((EP32,), jnp.int32),
        pltpu.VMEM((EP32,), jnp.int32),
        pltpu.VMEM((EP32,), jnp.float32),
        pltpu.VMEM((EP32,), jnp.float32),
    ],
)
def _norm_kernel(dinv_hbm, src_hbm, dst_hbm, ew_hbm, out_hbm,
                 dinvv, srcv, dstv, ewv, normv):
    wid = lax.axis_index("s") * 2 + lax.axis_index("c")
    pltpu.sync_copy(dinv_hbm, dinvv)
    pltpu.sync_copy(src_hbm.at[wid], srcv)
    pltpu.sync_copy(dst_hbm.at[wid], dstv)
    pltpu.sync_copy(ew_hbm.at[wid], ewv)

    def body(i, _):
        sl = pl.ds(i * 16, 16)
        sv = plsc.load_gather(dinvv, [srcv[sl]])
        tv = plsc.load_gather(dinvv, [dstv[sl]])
        normv[sl] = sv * tv * ewv[sl]
        return 0

    lax.fori_loop(0, EP32 // 16, body, 0)
    pltpu.sync_copy(normv, out_hbm.at[wid])


NCH16 = E_PAD // 16 // C
NCHH = NCH16 // 2
NHALF = N_PAD // 2
NACC = NHALF + C
NOUT = NHALF // 16


@functools.partial(
    pl.kernel,
    out_type=jax.ShapeDtypeStruct((N_PAD, D), jnp.float32),
    mesh=_MESH,
    compiler_params=_SC_PARAMS,
    scratch_types=[
        pltpu.VMEM((NCHH, C), jnp.int32),
        pltpu.VMEM((NCHH, C), jnp.int32),
        pltpu.VMEM((NCHH, C), jnp.float32),
        pltpu.VMEM((C, D), jnp.float32),
        pltpu.VMEM((C, D), jnp.float32),
        pltpu.VMEM_SHARED((NACC, D), jnp.float32),
        pltpu.SemaphoreType.DMA,
        pltpu.SemaphoreType.DMA,
        pltpu.SemaphoreType.DMA,
        pltpu.SemaphoreType.DMA,
    ],
)
def _hop_kernel(h_hbm, src_hbm, dst_hbm, norm_hbm, out_hbm,
                srcv, dstv, normv, rows0, rows1, acc, gs0, gs1, ss0, ss1):
    s = lax.axis_index("s")
    c = lax.axis_index("c")
    base = c * NHALF

    def zb(i, _):
        for q in range(D // 16):
            rows0[i, pl.ds(q * 16, 16)] = jnp.zeros((16,), jnp.float32)
        return 0

    lax.fori_loop(0, C, zb, 0)
    for t in range(NOUT // C):
        pltpu.sync_copy(rows0, acc.at[pl.ds(s * NOUT + t * C, C)])
    plsc.subcore_barrier()

    def scale(rows, j):
        def sg(g, _):
            nv = normv[j, pl.ds(g * 16, 16)]
            for l in range(16):
                bc = lax.broadcast(nv[l], (16,))
                e = g * 16 + l
                for q in range(D // 16):
                    sl = pl.ds(q * 16, 16)
                    rows[e, sl] = rows[e, sl] * bc
            return 0

        lax.fori_loop(0, C // 16, sg, 0)

    NPAIR = NCHH // 2
    for half in range(2):
        pltpu.sync_copy(src_hbm.at[s * 2 + half], srcv)
        pltpu.sync_copy(dst_hbm.at[s * 2 + half], dstv)
        pltpu.sync_copy(norm_hbm.at[s * 2 + half], normv)

        def remap(j, _):
            def rg(g, _):
                sl = pl.ds(g * 16, 16)
                dv = dstv[j, sl] - base
                ok = (dv >= 0) & (dv < NHALF)
                dstv[j, sl] = jnp.where(ok, dv, NHALF)
                return 0

            lax.fori_loop(0, C // 16, rg, 0)
            return 0

        lax.fori_loop(0, NCHH, remap, 0)

        pltpu.async_copy(h_hbm.at[srcv.at[0]], rows0, gs0)

        def pair(j2, _):
            a = j2 * 2
            pltpu.make_async_copy(h_hbm.at[srcv.at[a]], rows0, gs0).wait()

            @pl.when(j2 > 0)
            def _():
                pltpu.make_async_copy(rows1, acc.at[dstv.at[a - 1]], ss1).wait()

            pltpu.async_copy(h_hbm.at[srcv.at[a + 1]], rows1, gs1)
            scale(rows0, a)
            pltpu.async_copy(rows0, acc.at[dstv.at[a]], ss0, add=True)
            pltpu.make_async_copy(h_hbm.at[srcv.at[a + 1]], rows1, gs1).wait()

            @pl.when(j2 < NPAIR - 1)
            def _():
                pltpu.make_async_copy(rows0, acc.at[dstv.at[a]], ss0).wait()
                pltpu.async_copy(h_hbm.at[srcv.at[a + 2]], rows0, gs0)

            scale(rows1, a + 1)
            pltpu.async_copy(rows1, acc.at[dstv.at[a + 1]], ss1, add=True)
            return 0

        lax.fori_loop(0, NPAIR, pair, 0)
        pltpu.make_async_copy(rows0, acc.at[dstv.at[NCHH - 2]], ss0).wait()
        pltpu.make_async_copy(rows1, acc.at[dstv.at[NCHH - 1]], ss1).wait()

    plsc.subcore_barrier()
    pltpu.sync_copy(acc.at[pl.ds(s * NOUT, NOUT)],
                    out_hbm.at[pl.ds(base + s * NOUT, NOUT)])


ER = E // 128
EPR = E_PAD // 128


def _pad_body(s_ref, d_ref, w_ref, so_ref, do_ref, wo_ref):
    so_ref[...] = jnp.zeros_like(so_ref)
    do_ref[...] = jnp.zeros_like(do_ref)
    wo_ref[...] = jnp.zeros_like(wo_ref)
    so_ref[pl.ds(0, ER), :] = s_ref[...]
    do_ref[pl.ds(0, ER), :] = d_ref[...]
    wo_ref[pl.ds(0, ER), :] = w_ref[...]


def _pad_edges(s, d, w):
    return pl.pallas_call(
        _pad_body,
        out_shape=[
            jax.ShapeDtypeStruct((EPR, 128), jnp.int32),
            jax.ShapeDtypeStruct((EPR, 128), jnp.int32),
            jax.ShapeDtypeStruct((EPR, 128), jnp.float32),
        ],
    )(s.reshape(ER, 128), d.reshape(ER, 128), w.reshape(ER, 128))


def _combine_body(a_ref, dv_ref, h_ref, out_ref):
    dv = dv_ref[...]
    out_ref[...] = a_ref[...] + (dv * dv) * h_ref[...]


def _combine(a, dinv_col, h):
    blk = 2000
    return pl.pallas_call(
        _combine_body,
        grid=(N // blk,),
        in_specs=[
            pl.BlockSpec((blk, D), lambda i: (i, 0)),
            pl.BlockSpec((blk, 1), lambda i: (i, 0)),
            pl.BlockSpec((blk, D), lambda i: (i, 0)),
        ],
        out_specs=pl.BlockSpec((blk, D), lambda i: (i, 0)),
        out_shape=jax.ShapeDtypeStruct((N, D), jnp.float32),
    )(a, dinv_col, h)


def _head_body(q_ref, dv_ref, h1_ref, w_ref, b_ref, out_ref):
    dv = dv_ref[...]
    h = q_ref[...] + (dv * dv) * h1_ref[...]
    z = lax.dot_general(h, w_ref[...], (((1,), (1,)), ((), ())),
                        preferred_element_type=jnp.float32)
    z = z + b_ref[...]
    z = jnp.maximum(z, 0.0)
    m = jnp.max(z, axis=-1, keepdims=True)
    zs = z - m
    lse = jnp.log(jnp.sum(jnp.exp(zs), axis=-1, keepdims=True))
    out_ref[...] = zs - lse


def _head(q, dinv_col, h1, W, b2):
    blk = 2000
    return pl.pallas_call(
        _head_body,
        grid=(N // blk,),
        in_specs=[
            pl.BlockSpec((blk, D), lambda i: (i, 0)),
            pl.BlockSpec((blk, 1), lambda i: (i, 0)),
            pl.BlockSpec((blk, D), lambda i: (i, 0)),
            pl.BlockSpec((D, D), lambda i: (0, 0)),
            pl.BlockSpec((1, D), lambda i: (0, 0)),
        ],
        out_specs=pl.BlockSpec((blk, D), lambda i: (i, 0)),
        out_shape=jax.ShapeDtypeStruct((N, D), jnp.float32),
    )(q, dinv_col, h1, W, b2)


def kernel(x, edge_index, edge_attr, W, b):
    srcf, dstf, ewf = _pad_edges(edge_index[0].astype(jnp.int32),
                                 edge_index[1].astype(jnp.int32),
                                 edge_attr.astype(jnp.float32))

    src32 = srcf.reshape(32, EP32)
    dst32 = dstf.reshape(32, EP32)
    ew32 = ewf.reshape(32, EP32)

    degp = _deg_kernel(dst32, ew32)
    dinv = _dinv(degp.reshape(32, N_PAD // 128, 128)).reshape(N_PAD)
    norm = _norm_kernel(dinv, src32, dst32, ew32)

    srcc = srcf.reshape(32, NCHH, C)
    dstc = dstf.reshape(32, NCHH, C)
    normc = norm.reshape(32, NCHH, C)

    dinv_col = dinv.reshape(N_PAD, 1)[:N]
    h = x.astype(jnp.float32)
    p = _hop_kernel(h, srcc, dstc, normc)
    h1 = _combine(p[:N], dinv_col, h)
    q = _hop_kernel(h1, srcc, dstc, normc)

    return _head(q[:N], dinv_col, h1, W.astype(jnp.float32),
                 b.astype(jnp.float32).reshape(1, D))
